# baseline (speedup 1.0000x reference)
import jax
import jax.numpy as jnp
from jax.experimental import pallas as pl

_NUM_REGIONS = 17
_FEAT = 7
_ROWS = 82
_COLS = 67
_CELLS_PER_REGION = 20
_NCOORD = _NUM_REGIONS * _CELLS_PER_REGION
_KPAD = 128
_RPAD = 88
_CBLK = 8
_MCOL = _FEAT * _RPAD
_M = _CBLK * _MCOL


def _gridding_kernel(coords_ref, xt_ref, out_ref):
    col0 = pl.program_id(0) * _CBLK
    shape4 = (_CBLK, _FEAT, _RPAD, 1)
    c_loc = jax.lax.broadcasted_iota(jnp.int32, shape4, 0).reshape(_M, 1)
    f = jax.lax.broadcasted_iota(jnp.int32, shape4, 1).reshape(_M, 1)
    row = jax.lax.broadcasted_iota(jnp.int32, shape4, 2).reshape(_M, 1)
    r_i = coords_ref[0:1, :]
    c_i = coords_ref[1:2, :]
    key_i = c_i * _RPAD + r_i
    key_m = (col0 + c_loc) * _RPAD + row
    mask = (key_m == key_i).astype(jnp.float32)
    reg_s = jax.lax.broadcasted_iota(jnp.int32, (_NCOORD, _KPAD), 0) // _CELLS_PER_REGION
    kdiv7 = jax.lax.broadcasted_iota(jnp.int32, (_NCOORD, _KPAD), 1) // _FEAT
    p = (reg_s == kdiv7).astype(jnp.float32)
    w0 = jax.lax.dot(mask, p, precision=jax.lax.Precision.DEFAULT,
                     preferred_element_type=jnp.float32)
    lane7 = jax.lax.broadcasted_iota(jnp.int32, (1, _KPAD), 1)
    lane7 = lane7 - (lane7 // _FEAT) * _FEAT
    w = w0 * (lane7 == f).astype(jnp.float32)
    res = jax.lax.dot(
        w, xt_ref[...], precision=jax.lax.Precision.DEFAULT,
        preferred_element_type=jnp.float32)
    res4 = res.reshape(_CBLK, _FEAT, _RPAD, res.shape[1])
    out_ref[...] = jax.lax.slice(
        res4, (0, 0, 0, 0), (_CBLK, _FEAT, _ROWS, res.shape[1]))


def kernel(x, coords):
    b = x.shape[0]
    xt = jnp.pad(x.T, ((0, _KPAD - x.shape[1]), (0, 0)))
    coords_t = coords.T
    out = pl.pallas_call(
        _gridding_kernel,
        grid=(pl.cdiv(_COLS, _CBLK),),
        in_specs=[
            pl.BlockSpec((2, _NCOORD), lambda i: (0, 0)),
            pl.BlockSpec((_KPAD, b), lambda i: (0, 0)),
        ],
        out_specs=pl.BlockSpec((_CBLK, _FEAT, _ROWS, b), lambda i: (i, 0, 0, 0)),
        out_shape=jax.ShapeDtypeStruct((_COLS, _FEAT, _ROWS, b), x.dtype),
    )(coords_t, xt)
    return out.transpose(3, 2, 0, 1)


# speedup vs baseline: 1.0353x; 1.0353x over previous
import jax
import jax.numpy as jnp
from jax.experimental import pallas as pl

_NUM_REGIONS = 17
_FEAT = 7
_ROWS = 82
_COLS = 67
_CELLS_PER_REGION = 20
_NCOORD = _NUM_REGIONS * _CELLS_PER_REGION
_KPAD = 128
_RPAD = 88
_CBLK = 4
_MCOL = _FEAT * _RPAD
_M = _CBLK * _MCOL


def _gridding_kernel(coords_ref, xt_ref, out_ref):
    col0 = pl.program_id(0) * _CBLK
    shape4 = (_CBLK, _FEAT, _RPAD, 1)
    c_loc = jax.lax.broadcasted_iota(jnp.int32, shape4, 0).reshape(_M, 1)
    f = jax.lax.broadcasted_iota(jnp.int32, shape4, 1).reshape(_M, 1)
    row = jax.lax.broadcasted_iota(jnp.int32, shape4, 2).reshape(_M, 1)
    r_i = coords_ref[0:1, :]
    c_i = coords_ref[1:2, :]
    key_i = c_i * _RPAD + r_i
    key_m = (col0 + c_loc) * _RPAD + row
    mask = (key_m == key_i).astype(jnp.float32)
    reg_s = jax.lax.broadcasted_iota(jnp.int32, (_NCOORD, _KPAD), 0) // _CELLS_PER_REGION
    kdiv7 = jax.lax.broadcasted_iota(jnp.int32, (_NCOORD, _KPAD), 1) // _FEAT
    p = (reg_s == kdiv7).astype(jnp.float32)
    w0 = jax.lax.dot(mask, p, precision=jax.lax.Precision.DEFAULT,
                     preferred_element_type=jnp.float32)
    lane7 = jax.lax.broadcasted_iota(jnp.int32, (1, _KPAD), 1)
    lane7 = lane7 - (lane7 // _FEAT) * _FEAT
    w = w0 * (lane7 == f).astype(jnp.float32)
    res = jax.lax.dot(
        w, xt_ref[...], precision=jax.lax.Precision.DEFAULT,
        preferred_element_type=jnp.float32)
    res4 = res.reshape(_CBLK, _FEAT, _RPAD, res.shape[1])
    out_ref[...] = jax.lax.slice(
        res4, (0, 0, 0, 0), (_CBLK, _FEAT, _ROWS, res.shape[1]))


def kernel(x, coords):
    b = x.shape[0]
    xt = jnp.pad(x.T, ((0, _KPAD - x.shape[1]), (0, 0)))
    coords_t = coords.T
    out = pl.pallas_call(
        _gridding_kernel,
        grid=(pl.cdiv(_COLS, _CBLK),),
        in_specs=[
            pl.BlockSpec((2, _NCOORD), lambda i: (0, 0)),
            pl.BlockSpec((_KPAD, b), lambda i: (0, 0)),
        ],
        out_specs=pl.BlockSpec((_CBLK, _FEAT, _ROWS, b), lambda i: (i, 0, 0, 0)),
        out_shape=jax.ShapeDtypeStruct((_COLS, _FEAT, _ROWS, b), x.dtype),
    )(coords_t, xt)
    return out.transpose(3, 2, 0, 1)
